# Initial kernel scaffold; baseline (speedup 1.0000x reference)
import functools

import jax
import jax.numpy as jnp
from jax import lax
from jax.experimental import pallas as pl
from jax.experimental.pallas import tpu as pltpu
from jax.experimental.pallas import tpu_sc as plsc

HEADS = 4
H = 256
EDIM = 128
NEG = 0.2

N = 10000
NP = 10240
DIM = 128
NC1 = 8
CW1 = 128
RW1 = 144
NC2 = 2
CW2 = 64
RW2 = 80

NSC = 2
NTEC = 16
NW = NSC * NTEC
L = 16

EB = 128

_mesh = plsc.VectorSubcoreMesh(core_axis_name="c", subcore_axis_name="s")


def _elu(x):
    return jnp.where(x > 0, x, jnp.exp(x) - 1.0)


def _mm1_body(x_ref, w_ref, as_ref, ad_ref, hch_ref, at_ref, dt_ref):
    c = pl.program_id(1)
    h = jnp.dot(x_ref[...], w_ref[...], preferred_element_type=jnp.float32)
    hch_ref[0] = h
    pa = jnp.sum(h * as_ref[...], axis=-1).reshape(1, -1)
    pd = jnp.sum(h * ad_ref[...], axis=-1).reshape(1, -1)

    @pl.when(c % 2 == 0)
    def _():
        at_ref[...] = pa
        dt_ref[...] = pd

    @pl.when(c % 2 != 0)
    def _():
        at_ref[...] += pa
        dt_ref[...] += pd


def _k_mm1(xp, W1, a1s, a1d):
    nb = NP // 512
    return pl.pallas_call(
        _mm1_body,
        grid=(nb, NC1),
        in_specs=[
            pl.BlockSpec((512, DIM), lambda n, c: (n, 0)),
            pl.BlockSpec((DIM, CW1), lambda n, c: (0, c)),
            pl.BlockSpec((1, CW1), lambda n, c: (c, 0)),
            pl.BlockSpec((1, CW1), lambda n, c: (c, 0)),
        ],
        out_specs=[
            pl.BlockSpec((1, 512, CW1), lambda n, c: (c, n, 0)),
            pl.BlockSpec((1, 512), lambda n, c: (c // 2, n)),
            pl.BlockSpec((1, 512), lambda n, c: (c // 2, n)),
        ],
        out_shape=[
            jax.ShapeDtypeStruct((NC1, NP, CW1), jnp.float32),
            jax.ShapeDtypeStruct((HEADS, NP), jnp.float32),
            jax.ShapeDtypeStruct((HEADS, NP), jnp.float32),
        ],
    )(xp, W1, a1s, a1d)


def _mm2_body(acc_ref, b1_ref, w2_ref, as2_ref, ad2_ref,
              glo_ref, ghi_ref, at_ref, dt_ref):
    c = pl.program_id(1)
    blk = acc_ref[0]
    num = blk[:, :CW1]
    den = jnp.maximum(blk[:, CW1:CW1 + 1], 1e-30)
    h1 = _elu(num / den + b1_ref[...])
    g = jnp.dot(h1, w2_ref[0], preferred_element_type=jnp.float32)

    @pl.when(c == 0)
    def _():
        glo_ref[...] = g[:, :CW2]
        ghi_ref[...] = g[:, CW2:]

    @pl.when(c != 0)
    def _():
        glo_ref[...] += g[:, :CW2]
        ghi_ref[...] += g[:, CW2:]

    @pl.when(c == NC1 - 1)
    def _():
        z = jnp.concatenate([glo_ref[...], ghi_ref[...]], axis=1)
        at_ref[...] = jnp.sum(z * as2_ref[...], axis=-1).reshape(1, -1)
        dt_ref[...] = jnp.sum(z * ad2_ref[...], axis=-1).reshape(1, -1)


def _k_mm2(acc1, b1r, W2r, a2s, a2d):
    nb = NP // 512
    return pl.pallas_call(
        _mm2_body,
        grid=(nb, NC1),
        in_specs=[
            pl.BlockSpec((1, 512, RW1), lambda n, c: (c, n, 0)),
            pl.BlockSpec((1, CW1), lambda n, c: (c, 0)),
            pl.BlockSpec((1, CW1, EDIM), lambda n, c: (c, 0, 0)),
            pl.BlockSpec((1, EDIM), lambda n, c: (0, 0)),
            pl.BlockSpec((1, EDIM), lambda n, c: (0, 0)),
        ],
        out_specs=[
            pl.BlockSpec((512, CW2), lambda n, c: (n, 0)),
            pl.BlockSpec((512, CW2), lambda n, c: (n, 0)),
            pl.BlockSpec((1, 512), lambda n, c: (0, n)),
            pl.BlockSpec((1, 512), lambda n, c: (0, n)),
        ],
        out_shape=[
            jax.ShapeDtypeStruct((NP, CW2), jnp.float32),
            jax.ShapeDtypeStruct((NP, CW2), jnp.float32),
            jax.ShapeDtypeStruct((1, NP), jnp.float32),
            jax.ShapeDtypeStruct((1, NP), jnp.float32),
        ],
    )(acc1, b1r, W2r, a2s, a2d)


def _fin_body(acc_ref, b2_ref, wc_ref, bc_ref, z_ref, lg_ref):
    lo = acc_ref[0]
    hi = acc_ref[1]
    zlo = lo[:, :CW2] / jnp.maximum(lo[:, CW2:CW2 + 1], 1e-30)
    zhi = hi[:, :CW2] / jnp.maximum(hi[:, CW2:CW2 + 1], 1e-30)
    z = _elu(jnp.concatenate([zlo, zhi], axis=1) + b2_ref[...])
    z_ref[...] = z
    lg_ref[...] = (jnp.dot(z, wc_ref[...], preferred_element_type=jnp.float32)
                   + bc_ref[...])


def _k_fin(acc2, b2r, Wcp, bcp):
    nb = NP // 512
    return pl.pallas_call(
        _fin_body,
        grid=(nb,),
        in_specs=[
            pl.BlockSpec((2, 512, RW2), lambda n: (0, n, 0)),
            pl.BlockSpec((1, EDIM), lambda n: (0, 0)),
            pl.BlockSpec((EDIM, 128), lambda n: (0, 0)),
            pl.BlockSpec((1, 128), lambda n: (0, 0)),
        ],
        out_specs=[
            pl.BlockSpec((512, EDIM), lambda n: (n, 0)),
            pl.BlockSpec((512, 128), lambda n: (n, 0)),
        ],
        out_shape=[
            jax.ShapeDtypeStruct((NP, EDIM), jnp.float32),
            jax.ShapeDtypeStruct((NP, 128), jnp.float32),
        ],
    )(acc2, b2r, Wcp, bcp)


def _wk_body(nheads, et, m, at_hbm, dt_hbm, src_hbm, dst_hbm, w_hbm,
             atv, dtv, srcv, dstv, wv, sem):
    wid = lax.axis_index("s") * NSC + lax.axis_index("c")
    base = wid * m
    pltpu.sync_copy(at_hbm, atv)
    pltpu.sync_copy(dt_hbm, dtv)
    pltpu.sync_copy(src_hbm.at[pl.ds(base, m)], srcv)
    pltpu.sync_copy(dst_hbm.at[pl.ds(base, m)], dstv)

    def body(g, _):
        s16 = srcv[pl.ds(g * L, L)]
        d16 = dstv[pl.ds(g * L, L)]
        eid = base + g * L + lax.iota(jnp.int32, L)
        ok = eid < et
        for h in range(nheads):
            a = plsc.load_gather(atv, [s16 + h * NP])
            b = plsc.load_gather(dtv, [d16 + h * NP])
            al = a + b
            al = jnp.where(al > 0, al, NEG * al)
            w = jnp.where(ok, jnp.exp(al), 0.0)
            wv[h, pl.ds(g * L, L)] = w
        return ()

    lax.fori_loop(0, m // L, body, (), unroll=4)
    for h in range(nheads):
        pltpu.sync_copy(wv.at[h], w_hbm.at[h, pl.ds(base, m)])


def _k_w(nheads, et, et_pad, at_flat, dt_flat, srcs, dsts):
    m = et_pad // NW
    kfn = functools.partial(
        pl.kernel,
        mesh=_mesh,
        out_type=jax.ShapeDtypeStruct((nheads, et_pad), jnp.float32),
        scratch_types=[
            pltpu.VMEM((nheads * NP,), jnp.float32),
            pltpu.VMEM((nheads * NP,), jnp.float32),
            pltpu.VMEM((m,), jnp.int32),
            pltpu.VMEM((m,), jnp.int32),
            pltpu.VMEM((nheads, m), jnp.float32),
            pltpu.SemaphoreType.DMA,
        ],
    )(functools.partial(_wk_body, nheads, et, m))
    return kfn(at_flat, dt_flat, srcs, dsts)


def _agg_body(ncpc, cph, cw, rw, et_pad, tbl_hbm, src_hbm, dst_hbm, w_hbm,
              out_hbm, srcv, dstv, idxv, wvv, rowv, stagedv, zv, acc_sh, sem):
    core = lax.axis_index("c")
    sid = lax.axis_index("s")
    mt = et_pad // NTEC
    nbat = mt // EB
    rows_per_tec = NP // NTEC

    zv[...] = jnp.zeros((EB, rw), jnp.float32)
    stagedv[...] = jnp.zeros((EB, rw), jnp.float32)

    for j in range(ncpc):
        cidx = core * ncpc + j
        head = cidx // cph

        def zbody(r, _):
            pltpu.sync_copy(
                zv, acc_sh.at[pl.ds(sid * rows_per_tec + r * EB, EB)])
            return ()
        lax.fori_loop(0, rows_per_tec // EB, zbody, ())
        plsc.subcore_barrier()

        def bbody(t, _):
            ebase = sid * mt + t * EB
            pltpu.sync_copy(src_hbm.at[pl.ds(ebase, EB)], srcv)
            pltpu.sync_copy(dst_hbm.at[pl.ds(ebase, EB)], dstv)
            pltpu.sync_copy(w_hbm.at[head, pl.ds(ebase, EB)], wvv)

            def ibody(g, _):
                idxv[pl.ds(g * L, L)] = srcv[pl.ds(g * L, L)] + cidx * NP
                return ()
            lax.fori_loop(0, EB // L, ibody, (), unroll=8)

            pltpu.async_copy(tbl_hbm.at[idxv], rowv, sem).wait()

            def ebody(i, _):
                w = wvv[i]
                for q in range(cw // L):
                    stagedv[i, pl.ds(q * L, L)] = rowv[i, pl.ds(q * L, L)] * w
                return ()
            lax.fori_loop(0, EB, ebody, (), unroll=2)

            def wbody(g, _):
                rows16 = lax.iota(jnp.int32, L) + g * L
                plsc.store_scatter(
                    stagedv, [rows16, jnp.full((L,), cw, jnp.int32)],
                    wvv[pl.ds(g * L, L)])
                return ()
            lax.fori_loop(0, EB // L, wbody, (), unroll=8)

            pltpu.sync_copy(stagedv, acc_sh.at[dstv], add=True)
            return ()

        lax.fori_loop(0, nbat, bbody, ())
        plsc.subcore_barrier()
        pltpu.sync_copy(
            acc_sh.at[pl.ds(sid * rows_per_tec, rows_per_tec)],
            out_hbm.at[cidx].at[pl.ds(sid * rows_per_tec, rows_per_tec)])
        plsc.subcore_barrier()


def _k_agg(nchunks, nheads, cw, rw, et_pad, tbl, srcs, dsts, w):
    ncpc = nchunks // NSC
    cph = nchunks // nheads
    kfn = functools.partial(
        pl.kernel,
        mesh=_mesh,
        out_type=jax.ShapeDtypeStruct((nchunks, NP, rw), jnp.float32),
        scratch_types=[
            pltpu.VMEM((EB,), jnp.int32),
            pltpu.VMEM((EB,), jnp.int32),
            pltpu.VMEM((EB,), jnp.int32),
            pltpu.VMEM((EB,), jnp.float32),
            pltpu.VMEM((EB, cw), jnp.float32),
            pltpu.VMEM((EB, rw), jnp.float32),
            pltpu.VMEM((EB, rw), jnp.float32),
            pltpu.VMEM_SHARED((NP, rw), jnp.float32),
            pltpu.SemaphoreType.DMA,
        ],
    )(functools.partial(_agg_body, ncpc, cph, cw, rw, et_pad))
    return kfn(tbl, srcs, dsts, w)


def _pred_body(pq_pad, z_hbm, i0_hbm, i1_hbm, out_hbm,
               i0v, i1v, av, bv, resv, sem):
    wid = lax.axis_index("s") * NSC + lax.axis_index("c")
    mp = pq_pad // NW
    base = wid * mp
    nbat = mp // EB

    def bbody(t, _):
        pbase = base + t * EB
        pltpu.sync_copy(i0_hbm.at[pl.ds(pbase, EB)], i0v)
        pltpu.sync_copy(i1_hbm.at[pl.ds(pbase, EB)], i1v)
        pltpu.async_copy(z_hbm.at[i0v], av, sem).wait()
        pltpu.async_copy(z_hbm.at[i1v], bv, sem).wait()

        def ebody(i, _):
            acc = av[i, pl.ds(0, L)] * bv[i, pl.ds(0, L)]
            for q in range(1, EDIM // L):
                acc = acc + av[i, pl.ds(q * L, L)] * bv[i, pl.ds(q * L, L)]
            resv[i] = jnp.sum(acc, axis=0)
            return ()
        lax.fori_loop(0, EB, ebody, (), unroll=2)

        def sbody(g, _):
            v = resv[pl.ds(g * L, L)]
            resv[pl.ds(g * L, L)] = 1.0 / (1.0 + jnp.exp(-v))
            return ()
        lax.fori_loop(0, EB // L, sbody, (), unroll=8)

        pltpu.sync_copy(resv, out_hbm.at[pl.ds(pbase, EB)])
        return ()

    lax.fori_loop(0, nbat, bbody, ())


def _k_pred(pq_pad, z, i0, i1):
    kfn = functools.partial(
        pl.kernel,
        mesh=_mesh,
        out_type=jax.ShapeDtypeStruct((pq_pad,), jnp.float32),
        scratch_types=[
            pltpu.VMEM((EB,), jnp.int32),
            pltpu.VMEM((EB,), jnp.int32),
            pltpu.VMEM((EB, EDIM), jnp.float32),
            pltpu.VMEM((EB, EDIM), jnp.float32),
            pltpu.VMEM((EB,), jnp.float32),
            pltpu.SemaphoreType.DMA,
        ],
    )(functools.partial(_pred_body, pq_pad))
    return kfn(z, i0, i1)


def kernel(x, e, p, n, W1, a_s1, a_d1, b1, W2, a_s2, a_d2, b2, Wc, bc):
    E = e.shape[1]
    P = p.shape[1]
    ET = E + N
    ET_pad = ((ET + NW * EB - 1) // (NW * EB)) * (NW * EB)
    PQ = 2 * P
    PQ_pad = ((PQ + NW * EB - 1) // (NW * EB)) * (NW * EB)

    xp = jnp.pad(x, ((0, NP - N), (0, 0)))
    loop = jnp.arange(N, dtype=e.dtype)
    ei = jnp.concatenate(
        [e, jnp.stack([loop, loop]),
         jnp.zeros((2, ET_pad - ET), e.dtype)], axis=1).astype(jnp.int32)
    srcs, dsts = ei[0], ei[1]
    pq = jnp.concatenate(
        [p, n, jnp.zeros((2, PQ_pad - PQ), p.dtype)], axis=1).astype(jnp.int32)
    a1s = a_s1.reshape(NC1, CW1)
    a1d = a_d1.reshape(NC1, CW1)
    b1r = b1.reshape(NC1, CW1)
    W2r = W2.reshape(NC1, CW1, EDIM)
    a2s = a_s2.reshape(1, EDIM)
    a2d = a_d2.reshape(1, EDIM)
    b2r = b2.reshape(1, EDIM)
    Wcp = jnp.pad(Wc, ((0, 0), (0, 128 - Wc.shape[1])))
    bcp = jnp.pad(bc, (0, 128 - bc.shape[0])).reshape(1, 128)

    hch, at1, dt1 = _k_mm1(xp, W1, a1s, a1d)
    w1 = _k_w(HEADS, ET, ET_pad, at1.reshape(-1), dt1.reshape(-1), srcs, dsts)
    acc1 = _k_agg(NC1, HEADS, CW1, RW1, ET_pad,
                  hch.reshape(NC1 * NP, CW1), srcs, dsts, w1)

    glo, ghi, at2, dt2 = _k_mm2(acc1, b1r, W2r, a2s, a2d)
    w2 = _k_w(1, ET, ET_pad, at2.reshape(-1), dt2.reshape(-1), srcs, dsts)
    g2s = jnp.concatenate([glo, ghi], axis=0)
    acc2 = _k_agg(NC2, 1, CW2, RW2, ET_pad, g2s, srcs, dsts, w2)

    zf, lg = _k_fin(acc2, b2r, Wcp, bcp)
    preds = _k_pred(PQ_pad, zf, pq[0], pq[1])

    z = zf[:N]
    logits = lg[:N, :Wc.shape[1]]
    return (z, logits, preds[:PQ])


# speedup vs baseline: 5.0398x; 5.0398x over previous
import functools

import jax
import jax.numpy as jnp
from jax import lax
from jax.experimental import pallas as pl
from jax.experimental.pallas import tpu as pltpu
from jax.experimental.pallas import tpu_sc as plsc

HEADS = 4
H = 256
EDIM = 128
NEG = 0.2

N = 10000
NP = 10240
DIM = 128
NC1 = 16
CW1 = 64
RW1 = 80
CPH1 = NC1 // HEADS
NC2 = 4
CW2 = 32
RW2 = 48

NSC = 2
NTEC = 16
NW = NSC * NTEC
L = 16

EB = 128

_mesh = plsc.VectorSubcoreMesh(core_axis_name="c", subcore_axis_name="s")


def _elu(x):
    return jnp.where(x > 0, x, jnp.exp(x) - 1.0)


def _mm1_body(x_ref, w_ref, as_ref, ad_ref, hch_ref, at_ref, dt_ref):
    c = pl.program_id(1)
    h = jnp.dot(x_ref[...], w_ref[0], preferred_element_type=jnp.float32)
    hch_ref[0] = h
    pa = jnp.sum(h * as_ref[0], axis=-1).reshape(1, 1, -1)
    pd = jnp.sum(h * ad_ref[0], axis=-1).reshape(1, 1, -1)

    @pl.when(c % CPH1 == 0)
    def _():
        at_ref[...] = pa
        dt_ref[...] = pd

    @pl.when(c % CPH1 != 0)
    def _():
        at_ref[...] += pa
        dt_ref[...] += pd


def _k_mm1(xp, W1, a1s, a1d):
    nb = NP // 512
    return pl.pallas_call(
        _mm1_body,
        grid=(nb, NC1),
        in_specs=[
            pl.BlockSpec((512, DIM), lambda n, c: (n, 0)),
            pl.BlockSpec((1, DIM, CW1), lambda n, c: (c, 0, 0)),
            pl.BlockSpec((1, 1, CW1), lambda n, c: (c, 0, 0)),
            pl.BlockSpec((1, 1, CW1), lambda n, c: (c, 0, 0)),
        ],
        out_specs=[
            pl.BlockSpec((1, 512, CW1), lambda n, c: (c, n, 0)),
            pl.BlockSpec((1, 1, 512), lambda n, c: (c // CPH1, 0, n)),
            pl.BlockSpec((1, 1, 512), lambda n, c: (c // CPH1, 0, n)),
        ],
        out_shape=[
            jax.ShapeDtypeStruct((NC1, NP, CW1), jnp.float32),
            jax.ShapeDtypeStruct((HEADS, 1, NP), jnp.float32),
            jax.ShapeDtypeStruct((HEADS, 1, NP), jnp.float32),
        ],
    )(xp, W1, a1s, a1d)


def _mm2_body(acc_ref, b1_ref, w2_ref, as2_ref, ad2_ref,
              g0_ref, g1_ref, g2_ref, g3_ref, at_ref, dt_ref):
    c = pl.program_id(1)
    blk = acc_ref[0]
    num = blk[:, :CW1]
    den = jnp.maximum(blk[:, CW1:CW1 + 1], 1e-30)
    h1 = _elu(num / den + b1_ref[0])
    g = jnp.dot(h1, w2_ref[0], preferred_element_type=jnp.float32)
    grefs = (g0_ref, g1_ref, g2_ref, g3_ref)

    @pl.when(c == 0)
    def _():
        for q, gr in enumerate(grefs):
            gr[...] = g[:, q * CW2:(q + 1) * CW2]

    @pl.when(c != 0)
    def _():
        for q, gr in enumerate(grefs):
            gr[...] += g[:, q * CW2:(q + 1) * CW2]

    @pl.when(c == NC1 - 1)
    def _():
        z = jnp.concatenate([gr[...] for gr in grefs], axis=1)
        at_ref[...] = jnp.sum(z * as2_ref[...], axis=-1).reshape(1, -1)
        dt_ref[...] = jnp.sum(z * ad2_ref[...], axis=-1).reshape(1, -1)


def _k_mm2(acc1, b1r, W2r, a2s, a2d):
    nb = NP // 512
    return pl.pallas_call(
        _mm2_body,
        grid=(nb, NC1),
        in_specs=[
            pl.BlockSpec((1, 512, RW1), lambda n, c: (c, n, 0)),
            pl.BlockSpec((1, 1, CW1), lambda n, c: (c, 0, 0)),
            pl.BlockSpec((1, CW1, EDIM), lambda n, c: (c, 0, 0)),
            pl.BlockSpec((1, EDIM), lambda n, c: (0, 0)),
            pl.BlockSpec((1, EDIM), lambda n, c: (0, 0)),
        ],
        out_specs=(
            [pl.BlockSpec((512, CW2), lambda n, c: (n, 0))] * NC2
            + [pl.BlockSpec((1, 512), lambda n, c: (0, n))] * 2
        ),
        out_shape=(
            [jax.ShapeDtypeStruct((NP, CW2), jnp.float32)] * NC2
            + [jax.ShapeDtypeStruct((1, NP), jnp.float32)] * 2
        ),
    )(acc1, b1r, W2r, a2s, a2d)


def _fin_body(acc_ref, b2_ref, wc_ref, bc_ref, z_ref, lg_ref):
    parts = []
    for q in range(NC2):
        blk = acc_ref[q]
        parts.append(blk[:, :CW2] / jnp.maximum(blk[:, CW2:CW2 + 1], 1e-30))
    z = _elu(jnp.concatenate(parts, axis=1) + b2_ref[...])
    z_ref[...] = z
    lg_ref[...] = (jnp.dot(z, wc_ref[...], preferred_element_type=jnp.float32)
                   + bc_ref[...])


def _k_fin(acc2, b2r, Wcp, bcp):
    nb = NP // 512
    return pl.pallas_call(
        _fin_body,
        grid=(nb,),
        in_specs=[
            pl.BlockSpec((NC2, 512, RW2), lambda n: (0, n, 0)),
            pl.BlockSpec((1, EDIM), lambda n: (0, 0)),
            pl.BlockSpec((EDIM, 128), lambda n: (0, 0)),
            pl.BlockSpec((1, 128), lambda n: (0, 0)),
        ],
        out_specs=[
            pl.BlockSpec((512, EDIM), lambda n: (n, 0)),
            pl.BlockSpec((512, 128), lambda n: (n, 0)),
        ],
        out_shape=[
            jax.ShapeDtypeStruct((NP, EDIM), jnp.float32),
            jax.ShapeDtypeStruct((NP, 128), jnp.float32),
        ],
    )(acc2, b2r, Wcp, bcp)


def _wk_body(nheads, et, m, at_hbm, dt_hbm, src_hbm, dst_hbm, w_hbm,
             atv, dtv, srcv, dstv, wv, sem):
    wid = lax.axis_index("s") * NSC + lax.axis_index("c")
    base = wid * m
    pltpu.sync_copy(at_hbm, atv)
    pltpu.sync_copy(dt_hbm, dtv)
    pltpu.sync_copy(src_hbm.at[pl.ds(base, m)], srcv)
    pltpu.sync_copy(dst_hbm.at[pl.ds(base, m)], dstv)

    def body(g, _):
        s16 = srcv[pl.ds(g * L, L)]
        d16 = dstv[pl.ds(g * L, L)]
        eid = base + g * L + lax.iota(jnp.int32, L)
        ok = eid < et
        for h in range(nheads):
            a = plsc.load_gather(atv, [s16 + h * NP])
            b = plsc.load_gather(dtv, [d16 + h * NP])
            al = a + b
            al = jnp.where(al > 0, al, NEG * al)
            w = jnp.where(ok, jnp.exp(al), 0.0)
            wv[h, pl.ds(g * L, L)] = w
        return ()

    lax.fori_loop(0, m // L, body, (), unroll=4)
    for h in range(nheads):
        pltpu.sync_copy(wv.at[h], w_hbm.at[h, pl.ds(base, m)])


def _k_w(nheads, et, et_pad, at_flat, dt_flat, srcs, dsts):
    m = et_pad // NW
    kfn = functools.partial(
        pl.kernel,
        mesh=_mesh,
        compiler_params=pltpu.CompilerParams(
            needs_layout_passes=False, use_tc_tiling_on_sc=False),
        out_type=jax.ShapeDtypeStruct((nheads, et_pad), jnp.float32),
        scratch_types=[
            pltpu.VMEM((nheads * NP,), jnp.float32),
            pltpu.VMEM((nheads * NP,), jnp.float32),
            pltpu.VMEM((m,), jnp.int32),
            pltpu.VMEM((m,), jnp.int32),
            pltpu.VMEM((nheads, m), jnp.float32),
            pltpu.SemaphoreType.DMA,
        ],
    )(functools.partial(_wk_body, nheads, et, m))
    return kfn(at_flat, dt_flat, srcs, dsts)


def _agg_body(ncpc, cph, cw, rw, et_pad, tbl_hbm, src_hbm, dst_hbm, w_hbm,
              out_hbm, srcv, dstv, idxv, wvv, rowv, stagedv, zv, acc_sh, sem):
    core = lax.axis_index("c")
    sid = lax.axis_index("s")
    mt = et_pad // NTEC
    nbat = mt // EB
    rows_per_tec = NP // NTEC

    def z0body(i, _):
        for q in range(rw // L):
            zv[i, pl.ds(q * L, L)] = jnp.zeros((L,), jnp.float32)
            stagedv[i, pl.ds(q * L, L)] = jnp.zeros((L,), jnp.float32)
        return ()
    lax.fori_loop(0, EB, z0body, ())

    for j in range(ncpc):
        cidx = core * ncpc + j
        head = cidx // cph

        def zbody(r, _):
            pltpu.sync_copy(
                zv, acc_sh.at[pl.ds(sid * rows_per_tec + r * EB, EB)])
            return ()
        lax.fori_loop(0, rows_per_tec // EB, zbody, ())
        plsc.subcore_barrier()

        def bbody(t, _):
            ebase = sid * mt + t * EB
            pltpu.sync_copy(src_hbm.at[pl.ds(ebase, EB)], srcv)
            pltpu.sync_copy(dst_hbm.at[pl.ds(ebase, EB)], dstv)
            pltpu.sync_copy(w_hbm.at[head, pl.ds(ebase, EB)], wvv)

            def ibody(g, _):
                idxv[pl.ds(g * L, L)] = srcv[pl.ds(g * L, L)] + cidx * NP
                return ()
            lax.fori_loop(0, EB // L, ibody, (), unroll=8)

            pltpu.async_copy(tbl_hbm.at[idxv], rowv, sem).wait()

            def gbody(g, _):
                w16 = wvv[pl.ds(g * L, L)]
                for lane in range(L):
                    i = g * L + lane
                    sp = plsc.load_gather(wvv, [jnp.full((L,), i, jnp.int32)])
                    for q in range(cw // L):
                        stagedv[i, pl.ds(q * L, L)] = (
                            rowv[i, pl.ds(q * L, L)] * sp)
                rows16 = lax.iota(jnp.int32, L) + g * L
                plsc.store_scatter(
                    stagedv, [rows16, jnp.full((L,), cw, jnp.int32)], w16)
                return ()
            lax.fori_loop(0, EB // L, gbody, ())

            pltpu.sync_copy(stagedv, acc_sh.at[dstv], add=True)
            return ()

        lax.fori_loop(0, nbat, bbody, ())
        plsc.subcore_barrier()
        pltpu.sync_copy(
            acc_sh.at[pl.ds(sid * rows_per_tec, rows_per_tec)],
            out_hbm.at[cidx].at[pl.ds(sid * rows_per_tec, rows_per_tec)])
        plsc.subcore_barrier()


def _k_agg(nchunks, nheads, cw, rw, et_pad, tbl, srcs, dsts, w):
    ncpc = nchunks // NSC
    cph = nchunks // nheads
    kfn = functools.partial(
        pl.kernel,
        mesh=_mesh,
        compiler_params=pltpu.CompilerParams(
            needs_layout_passes=False, use_tc_tiling_on_sc=False),
        out_type=jax.ShapeDtypeStruct((nchunks, NP, rw), jnp.float32),
        scratch_types=[
            pltpu.VMEM((EB,), jnp.int32),
            pltpu.VMEM((EB,), jnp.int32),
            pltpu.VMEM((EB,), jnp.int32),
            pltpu.VMEM((EB,), jnp.float32),
            pltpu.VMEM((EB, cw), jnp.float32),
            pltpu.VMEM((EB, rw), jnp.float32),
            pltpu.VMEM((EB, rw), jnp.float32),
            pltpu.VMEM_SHARED((NP, rw), jnp.float32),
            pltpu.SemaphoreType.DMA,
        ],
    )(functools.partial(_agg_body, ncpc, cph, cw, rw, et_pad))
    return kfn(tbl, srcs, dsts, w)


def _pred_body(pq_pad, z_hbm, i0_hbm, i1_hbm, out_hbm,
               i0v, i1v, av, bv, resv, sem):
    wid = lax.axis_index("s") * NSC + lax.axis_index("c")
    mp = pq_pad // NW
    base = wid * mp
    nbat = mp // EB

    def bbody(t, _):
        pbase = base + t * EB
        pltpu.sync_copy(i0_hbm.at[pl.ds(pbase, EB)], i0v)
        pltpu.sync_copy(i1_hbm.at[pl.ds(pbase, EB)], i1v)
        pltpu.async_copy(z_hbm.at[i0v], av, sem).wait()
        pltpu.async_copy(z_hbm.at[i1v], bv, sem).wait()

        lanes = lax.iota(jnp.int32, L)

        def gbody(g, _):
            res = jnp.zeros((L,), jnp.float32)
            for lane in range(L):
                i = g * L + lane
                acc = av[i, pl.ds(0, L)] * bv[i, pl.ds(0, L)]
                for q in range(1, EDIM // L):
                    acc = acc + av[i, pl.ds(q * L, L)] * bv[i, pl.ds(q * L, L)]
                s = jnp.sum(acc, axis=0)
                res = jnp.where(lanes == lane, s, res)
            resv[pl.ds(g * L, L)] = 1.0 / (1.0 + jnp.exp(-res))
            return ()
        lax.fori_loop(0, EB // L, gbody, ())

        pltpu.sync_copy(resv, out_hbm.at[pl.ds(pbase, EB)])
        return ()

    lax.fori_loop(0, nbat, bbody, ())


def _k_pred(pq_pad, z, i0, i1):
    kfn = functools.partial(
        pl.kernel,
        mesh=_mesh,
        compiler_params=pltpu.CompilerParams(
            needs_layout_passes=False, use_tc_tiling_on_sc=False),
        out_type=jax.ShapeDtypeStruct((pq_pad,), jnp.float32),
        scratch_types=[
            pltpu.VMEM((EB,), jnp.int32),
            pltpu.VMEM((EB,), jnp.int32),
            pltpu.VMEM((EB, EDIM), jnp.float32),
            pltpu.VMEM((EB, EDIM), jnp.float32),
            pltpu.VMEM((EB,), jnp.float32),
            pltpu.SemaphoreType.DMA,
        ],
    )(functools.partial(_pred_body, pq_pad))
    return kfn(z, i0, i1)


def kernel(x, e, p, n, W1, a_s1, a_d1, b1, W2, a_s2, a_d2, b2, Wc, bc):
    E = e.shape[1]
    P = p.shape[1]
    ET = E + N
    ET_pad = ((ET + NW * EB - 1) // (NW * EB)) * (NW * EB)
    PQ = 2 * P
    PQ_pad = ((PQ + NW * EB - 1) // (NW * EB)) * (NW * EB)

    xp = jnp.pad(x, ((0, NP - N), (0, 0)))
    loop = jnp.arange(N, dtype=e.dtype)
    ei = jnp.concatenate(
        [e, jnp.stack([loop, loop]),
         jnp.zeros((2, ET_pad - ET), e.dtype)], axis=1).astype(jnp.int32)
    srcs, dsts = ei[0], ei[1]
    pq = jnp.concatenate(
        [p, n, jnp.zeros((2, PQ_pad - PQ), p.dtype)], axis=1).astype(jnp.int32)
    a1s = a_s1.reshape(NC1, 1, CW1)
    a1d = a_d1.reshape(NC1, 1, CW1)
    b1r = b1.reshape(NC1, 1, CW1)
    W1r = W1.reshape(DIM, NC1, CW1).transpose(1, 0, 2)
    W2r = W2.reshape(NC1, CW1, EDIM)
    a2s = a_s2.reshape(1, EDIM)
    a2d = a_d2.reshape(1, EDIM)
    b2r = b2.reshape(1, EDIM)
    Wcp = jnp.pad(Wc, ((0, 0), (0, 128 - Wc.shape[1])))
    bcp = jnp.pad(bc, (0, 128 - bc.shape[0])).reshape(1, 128)

    hch, at1, dt1 = _k_mm1(xp, W1r, a1s, a1d)
    w1 = _k_w(HEADS, ET, ET_pad, at1.reshape(-1), dt1.reshape(-1), srcs, dsts)
    acc1 = _k_agg(NC1, HEADS, CW1, RW1, ET_pad,
                  hch.reshape(NC1 * NP, CW1), srcs, dsts, w1)

    g0, g1, g2, g3, at2, dt2 = _k_mm2(acc1, b1r, W2r, a2s, a2d)
    w2 = _k_w(1, ET, ET_pad, at2.reshape(-1), dt2.reshape(-1), srcs, dsts)
    g2s = jnp.concatenate([g0, g1, g2, g3], axis=0)
    acc2 = _k_agg(NC2, 1, CW2, RW2, ET_pad, g2s, srcs, dsts, w2)

    zf, lg = _k_fin(acc2, b2r, Wcp, bcp)
    preds = _k_pred(PQ_pad, zf, pq[0], pq[1])

    z = zf[:N]
    logits = lg[:N, :Wc.shape[1]]
    return (z, logits, preds[:PQ])


# speedup vs baseline: 7.8896x; 1.5655x over previous
import functools

import jax
import jax.numpy as jnp
from jax import lax
from jax.experimental import pallas as pl
from jax.experimental.pallas import tpu as pltpu
from jax.experimental.pallas import tpu_sc as plsc

HEADS = 4
H = 256
EDIM = 128
NEG = 0.2

N = 10000
NP = 10240
DIM = 128
NC1 = 16
CW1 = 64
RW1 = 80
CPH1 = NC1 // HEADS
NC2 = 4
CW2 = 32
RW2 = 48

NSC = 2
NTEC = 16
NW = NSC * NTEC
L = 16

EB = 128

_mesh = plsc.VectorSubcoreMesh(core_axis_name="c", subcore_axis_name="s")


def _elu(x):
    return jnp.where(x > 0, x, jnp.exp(x) - 1.0)


def _mm1_body(x_ref, w_ref, as_ref, ad_ref, hch_ref, at_ref, dt_ref):
    c = pl.program_id(1)
    h = jnp.dot(x_ref[...], w_ref[0], preferred_element_type=jnp.float32)
    hch_ref[0] = h
    pa = jnp.sum(h * as_ref[0], axis=-1).reshape(1, 1, -1)
    pd = jnp.sum(h * ad_ref[0], axis=-1).reshape(1, 1, -1)

    @pl.when(c % CPH1 == 0)
    def _():
        at_ref[...] = pa
        dt_ref[...] = pd

    @pl.when(c % CPH1 != 0)
    def _():
        at_ref[...] += pa
        dt_ref[...] += pd


def _k_mm1(xp, W1, a1s, a1d):
    nb = NP // 512
    return pl.pallas_call(
        _mm1_body,
        grid=(nb, NC1),
        in_specs=[
            pl.BlockSpec((512, DIM), lambda n, c: (n, 0)),
            pl.BlockSpec((1, DIM, CW1), lambda n, c: (c, 0, 0)),
            pl.BlockSpec((1, 1, CW1), lambda n, c: (c, 0, 0)),
            pl.BlockSpec((1, 1, CW1), lambda n, c: (c, 0, 0)),
        ],
        out_specs=[
            pl.BlockSpec((1, 512, CW1), lambda n, c: (c, n, 0)),
            pl.BlockSpec((1, 1, 512), lambda n, c: (c // CPH1, 0, n)),
            pl.BlockSpec((1, 1, 512), lambda n, c: (c // CPH1, 0, n)),
        ],
        out_shape=[
            jax.ShapeDtypeStruct((NC1, NP, CW1), jnp.float32),
            jax.ShapeDtypeStruct((HEADS, 1, NP), jnp.float32),
            jax.ShapeDtypeStruct((HEADS, 1, NP), jnp.float32),
        ],
    )(xp, W1, a1s, a1d)


def _mm2_body(acc_ref, b1_ref, w2_ref, as2_ref, ad2_ref,
              g0_ref, g1_ref, g2_ref, g3_ref, at_ref, dt_ref):
    c = pl.program_id(1)
    blk = acc_ref[0]
    num = blk[:, :CW1]
    den = jnp.maximum(blk[:, CW1:CW1 + 1], 1e-30)
    h1 = _elu(num / den + b1_ref[0])
    g = jnp.dot(h1, w2_ref[0], preferred_element_type=jnp.float32)
    grefs = (g0_ref, g1_ref, g2_ref, g3_ref)

    @pl.when(c == 0)
    def _():
        for q, gr in enumerate(grefs):
            gr[...] = g[:, q * CW2:(q + 1) * CW2]

    @pl.when(c != 0)
    def _():
        for q, gr in enumerate(grefs):
            gr[...] += g[:, q * CW2:(q + 1) * CW2]

    @pl.when(c == NC1 - 1)
    def _():
        z = jnp.concatenate([gr[...] for gr in grefs], axis=1)
        at_ref[...] = jnp.sum(z * as2_ref[...], axis=-1).reshape(1, -1)
        dt_ref[...] = jnp.sum(z * ad2_ref[...], axis=-1).reshape(1, -1)


def _k_mm2(acc1, b1r, W2r, a2s, a2d):
    nb = NP // 512
    return pl.pallas_call(
        _mm2_body,
        grid=(nb, NC1),
        in_specs=[
            pl.BlockSpec((1, 512, RW1), lambda n, c: (c, n, 0)),
            pl.BlockSpec((1, 1, CW1), lambda n, c: (c, 0, 0)),
            pl.BlockSpec((1, CW1, EDIM), lambda n, c: (c, 0, 0)),
            pl.BlockSpec((1, EDIM), lambda n, c: (0, 0)),
            pl.BlockSpec((1, EDIM), lambda n, c: (0, 0)),
        ],
        out_specs=(
            [pl.BlockSpec((512, CW2), lambda n, c: (n, 0))] * NC2
            + [pl.BlockSpec((1, 512), lambda n, c: (0, n))] * 2
        ),
        out_shape=(
            [jax.ShapeDtypeStruct((NP, CW2), jnp.float32)] * NC2
            + [jax.ShapeDtypeStruct((1, NP), jnp.float32)] * 2
        ),
    )(acc1, b1r, W2r, a2s, a2d)


def _fin_body(acc_ref, b2_ref, wc_ref, bc_ref, z_ref, lg_ref):
    parts = []
    for q in range(NC2):
        blk = acc_ref[q]
        parts.append(blk[:, :CW2] / jnp.maximum(blk[:, CW2:CW2 + 1], 1e-30))
    z = _elu(jnp.concatenate(parts, axis=1) + b2_ref[...])
    z_ref[...] = z
    lg_ref[...] = (jnp.dot(z, wc_ref[...], preferred_element_type=jnp.float32)
                   + bc_ref[...])


def _k_fin(acc2, b2r, Wcp, bcp):
    nb = NP // 512
    return pl.pallas_call(
        _fin_body,
        grid=(nb,),
        in_specs=[
            pl.BlockSpec((NC2, 512, RW2), lambda n: (0, n, 0)),
            pl.BlockSpec((1, EDIM), lambda n: (0, 0)),
            pl.BlockSpec((EDIM, 128), lambda n: (0, 0)),
            pl.BlockSpec((1, 128), lambda n: (0, 0)),
        ],
        out_specs=[
            pl.BlockSpec((512, EDIM), lambda n: (n, 0)),
            pl.BlockSpec((512, 128), lambda n: (n, 0)),
        ],
        out_shape=[
            jax.ShapeDtypeStruct((NP, EDIM), jnp.float32),
            jax.ShapeDtypeStruct((NP, 128), jnp.float32),
        ],
    )(acc2, b2r, Wcp, bcp)


def _wk_body(nheads, et, m, at_hbm, dt_hbm, src_hbm, dst_hbm, w_hbm,
             atv, dtv, srcv, dstv, wv, sem):
    wid = lax.axis_index("s") * NSC + lax.axis_index("c")
    base = wid * m
    pltpu.sync_copy(at_hbm, atv)
    pltpu.sync_copy(dt_hbm, dtv)
    pltpu.sync_copy(src_hbm.at[pl.ds(base, m)], srcv)
    pltpu.sync_copy(dst_hbm.at[pl.ds(base, m)], dstv)

    def body(g, _):
        s16 = srcv[pl.ds(g * L, L)]
        d16 = dstv[pl.ds(g * L, L)]
        eid = base + g * L + lax.iota(jnp.int32, L)
        ok = eid < et
        for h in range(nheads):
            a = plsc.load_gather(atv, [s16 + h * NP])
            b = plsc.load_gather(dtv, [d16 + h * NP])
            al = a + b
            al = jnp.where(al > 0, al, NEG * al)
            w = jnp.where(ok, jnp.exp(al), 0.0)
            wv[h, pl.ds(g * L, L)] = w
        return ()

    lax.fori_loop(0, m // L, body, (), unroll=4)
    for h in range(nheads):
        pltpu.sync_copy(wv.at[h], w_hbm.at[h, pl.ds(base, m)])


def _k_w(nheads, et, et_pad, at_flat, dt_flat, srcs, dsts):
    m = et_pad // NW
    kfn = functools.partial(
        pl.kernel,
        mesh=_mesh,
        compiler_params=pltpu.CompilerParams(
            needs_layout_passes=False, use_tc_tiling_on_sc=False),
        out_type=jax.ShapeDtypeStruct((nheads, et_pad), jnp.float32),
        scratch_types=[
            pltpu.VMEM((nheads * NP,), jnp.float32),
            pltpu.VMEM((nheads * NP,), jnp.float32),
            pltpu.VMEM((m,), jnp.int32),
            pltpu.VMEM((m,), jnp.int32),
            pltpu.VMEM((nheads, m), jnp.float32),
            pltpu.SemaphoreType.DMA,
        ],
    )(functools.partial(_wk_body, nheads, et, m))
    return kfn(at_flat, dt_flat, srcs, dsts)


def _agg_body(ncpc, cph, nheads, cw, rw, et_pad, tbl_hbm, src_hbm, dst_hbm,
              w_hbm, out_hbm, srcall, dstall, wall, idx2, dstv, row2, stagedv,
              zv, acc_sh, semg0, semg1):
    core = lax.axis_index("c")
    sid = lax.axis_index("s")
    mt = et_pad // NTEC
    nbat = mt // EB
    rows_per_tec = NP // NTEC
    semg = (semg0, semg1)

    pltpu.sync_copy(src_hbm.at[pl.ds(sid * mt, mt)], srcall)
    pltpu.sync_copy(dst_hbm.at[pl.ds(sid * mt, mt)], dstall)

    def z0body(i, _):
        for q in range(rw // L):
            zv[i, pl.ds(q * L, L)] = jnp.zeros((L,), jnp.float32)
            stagedv[i, pl.ds(q * L, L)] = jnp.zeros((L,), jnp.float32)
        return ()
    lax.fori_loop(0, EB, z0body, ())

    def build_gather_idx(b, t, cidx):
        def ibody(g, _):
            idx2[b, pl.ds(g * L, L)] = (
                srcall[pl.ds(t * EB + g * L, L)] + cidx * NP)
            return ()
        lax.fori_loop(0, EB // L, ibody, (), unroll=8)

    for j in range(ncpc):
        cidx = core * ncpc + j
        head = cidx // cph
        pltpu.sync_copy(w_hbm.at[head, pl.ds(sid * mt, mt)], wall)

        def zbody(r, _):
            pltpu.sync_copy(
                zv, acc_sh.at[pl.ds(sid * rows_per_tec + r * EB, EB)])
            return ()
        lax.fori_loop(0, rows_per_tec // EB, zbody, ())
        plsc.subcore_barrier()

        build_gather_idx(0, 0, cidx)
        pltpu.async_copy(tbl_hbm.at[idx2.at[0]], row2.at[0], semg[0])

        def kbody(k, _):
            for b in range(2):
                t = 2 * k + b
                nb = 1 - b

                @pl.when(t + 1 < nbat)
                def _():
                    build_gather_idx(nb, t + 1, cidx)
                    pltpu.async_copy(
                        tbl_hbm.at[idx2.at[nb]], row2.at[nb], semg[nb])

                pltpu.make_async_copy(
                    tbl_hbm.at[idx2.at[b]], row2.at[b], semg[b]).wait()

                rowv = row2.at[b]

                def gbody(g, _):
                    w16 = wall[pl.ds(t * EB + g * L, L)]
                    dstv[pl.ds(g * L, L)] = dstall[pl.ds(t * EB + g * L, L)]
                    for lane in range(L):
                        i = g * L + lane
                        sp = plsc.load_gather(
                            wall,
                            [jnp.full((L,), t * EB + i, jnp.int32)])
                        for q in range(cw // L):
                            stagedv[i, pl.ds(q * L, L)] = (
                                rowv[i, pl.ds(q * L, L)] * sp)
                    rows16 = lax.iota(jnp.int32, L) + g * L
                    plsc.store_scatter(
                        stagedv, [rows16, jnp.full((L,), cw, jnp.int32)], w16)
                    return ()
                lax.fori_loop(0, EB // L, gbody, ())

                pltpu.sync_copy(stagedv, acc_sh.at[dstv], add=True)
            return ()

        lax.fori_loop(0, nbat // 2, kbody, ())
        plsc.subcore_barrier()
        pltpu.sync_copy(
            acc_sh.at[pl.ds(sid * rows_per_tec, rows_per_tec)],
            out_hbm.at[cidx].at[pl.ds(sid * rows_per_tec, rows_per_tec)])
        plsc.subcore_barrier()


def _k_agg(nchunks, nheads, cw, rw, et_pad, tbl, srcs, dsts, w):
    ncpc = nchunks // NSC
    cph = nchunks // nheads
    mt = et_pad // NTEC
    kfn = functools.partial(
        pl.kernel,
        mesh=_mesh,
        compiler_params=pltpu.CompilerParams(
            needs_layout_passes=False, use_tc_tiling_on_sc=False),
        out_type=jax.ShapeDtypeStruct((nchunks, NP, rw), jnp.float32),
        scratch_types=[
            pltpu.VMEM((mt,), jnp.int32),
            pltpu.VMEM((mt,), jnp.int32),
            pltpu.VMEM((mt,), jnp.float32),
            pltpu.VMEM((2, EB), jnp.int32),
            pltpu.VMEM((EB,), jnp.int32),
            pltpu.VMEM((2, EB, cw), jnp.float32),
            pltpu.VMEM((EB, rw), jnp.float32),
            pltpu.VMEM((EB, rw), jnp.float32),
            pltpu.VMEM_SHARED((NP, rw), jnp.float32),
            pltpu.SemaphoreType.DMA,
            pltpu.SemaphoreType.DMA,
        ],
    )(functools.partial(_agg_body, ncpc, cph, nheads, cw, rw, et_pad))
    return kfn(tbl, srcs, dsts, w)


def _pred_body(pq_pad, z_hbm, i0_hbm, i1_hbm, out_hbm,
               i0v, i1v, av, bv, resv, sem):
    wid = lax.axis_index("s") * NSC + lax.axis_index("c")
    mp = pq_pad // NW
    base = wid * mp
    nbat = mp // EB

    def bbody(t, _):
        pbase = base + t * EB
        pltpu.sync_copy(i0_hbm.at[pl.ds(pbase, EB)], i0v)
        pltpu.sync_copy(i1_hbm.at[pl.ds(pbase, EB)], i1v)
        pltpu.async_copy(z_hbm.at[i0v], av, sem).wait()
        pltpu.async_copy(z_hbm.at[i1v], bv, sem).wait()

        lanes = lax.iota(jnp.int32, L)

        def gbody(g, _):
            res = jnp.zeros((L,), jnp.float32)
            for lane in range(L):
                i = g * L + lane
                acc = av[i, pl.ds(0, L)] * bv[i, pl.ds(0, L)]
                for q in range(1, EDIM // L):
                    acc = acc + av[i, pl.ds(q * L, L)] * bv[i, pl.ds(q * L, L)]
                s = jnp.sum(acc, axis=0)
                res = jnp.where(lanes == lane, s, res)
            resv[pl.ds(g * L, L)] = 1.0 / (1.0 + jnp.exp(-res))
            return ()
        lax.fori_loop(0, EB // L, gbody, ())

        pltpu.sync_copy(resv, out_hbm.at[pl.ds(pbase, EB)])
        return ()

    lax.fori_loop(0, nbat, bbody, ())


def _k_pred(pq_pad, z, i0, i1):
    kfn = functools.partial(
        pl.kernel,
        mesh=_mesh,
        compiler_params=pltpu.CompilerParams(
            needs_layout_passes=False, use_tc_tiling_on_sc=False),
        out_type=jax.ShapeDtypeStruct((pq_pad,), jnp.float32),
        scratch_types=[
            pltpu.VMEM((EB,), jnp.int32),
            pltpu.VMEM((EB,), jnp.int32),
            pltpu.VMEM((EB, EDIM), jnp.float32),
            pltpu.VMEM((EB, EDIM), jnp.float32),
            pltpu.VMEM((EB,), jnp.float32),
            pltpu.SemaphoreType.DMA,
        ],
    )(functools.partial(_pred_body, pq_pad))
    return kfn(z, i0, i1)


def kernel(x, e, p, n, W1, a_s1, a_d1, b1, W2, a_s2, a_d2, b2, Wc, bc):
    E = e.shape[1]
    P = p.shape[1]
    ET = E + N
    ET_pad = ((ET + NW * EB - 1) // (NW * EB)) * (NW * EB)
    PQ = 2 * P
    PQ_pad = ((PQ + NW * EB - 1) // (NW * EB)) * (NW * EB)

    xp = jnp.pad(x, ((0, NP - N), (0, 0)))
    loop = jnp.arange(N, dtype=e.dtype)
    ei = jnp.concatenate(
        [e, jnp.stack([loop, loop]),
         jnp.zeros((2, ET_pad - ET), e.dtype)], axis=1).astype(jnp.int32)
    srcs, dsts = ei[0], ei[1]
    pq = jnp.concatenate(
        [p, n, jnp.zeros((2, PQ_pad - PQ), p.dtype)], axis=1).astype(jnp.int32)
    a1s = a_s1.reshape(NC1, 1, CW1)
    a1d = a_d1.reshape(NC1, 1, CW1)
    b1r = b1.reshape(NC1, 1, CW1)
    W1r = W1.reshape(DIM, NC1, CW1).transpose(1, 0, 2)
    W2r = W2.reshape(NC1, CW1, EDIM)
    a2s = a_s2.reshape(1, EDIM)
    a2d = a_d2.reshape(1, EDIM)
    b2r = b2.reshape(1, EDIM)
    Wcp = jnp.pad(Wc, ((0, 0), (0, 128 - Wc.shape[1])))
    bcp = jnp.pad(bc, (0, 128 - bc.shape[0])).reshape(1, 128)

    hch, at1, dt1 = _k_mm1(xp, W1r, a1s, a1d)
    w1 = _k_w(HEADS, ET, ET_pad, at1.reshape(-1), dt1.reshape(-1), srcs, dsts)
    acc1 = _k_agg(NC1, HEADS, CW1, RW1, ET_pad,
                  hch.reshape(NC1 * NP, CW1), srcs, dsts, w1)

    g0, g1, g2, g3, at2, dt2 = _k_mm2(acc1, b1r, W2r, a2s, a2d)
    w2 = _k_w(1, ET, ET_pad, at2.reshape(-1), dt2.reshape(-1), srcs, dsts)
    g2s = jnp.concatenate([g0, g1, g2, g3], axis=0)
    acc2 = _k_agg(NC2, 1, CW2, RW2, ET_pad, g2s, srcs, dsts, w2)

    zf, lg = _k_fin(acc2, b2r, Wcp, bcp)
    preds = _k_pred(PQ_pad, zf, pq[0], pq[1])

    z = zf[:N]
    logits = lg[:N, :Wc.shape[1]]
    return (z, logits, preds[:PQ])


# speedup vs baseline: 8.5340x; 1.0817x over previous
import functools

import jax
import jax.numpy as jnp
from jax import lax
from jax.experimental import pallas as pl
from jax.experimental.pallas import tpu as pltpu
from jax.experimental.pallas import tpu_sc as plsc

HEADS = 4
H = 256
EDIM = 128
NEG = 0.2

N = 10000
NP = 10240
DIM = 128
NC1 = 16
CW1 = 64
RW1 = 80
CPH1 = NC1 // HEADS
NC2 = 4
CW2 = 32
RW2 = 48

NSC = 2
NTEC = 16
NW = NSC * NTEC
L = 16

EB = 128

_mesh = plsc.VectorSubcoreMesh(core_axis_name="c", subcore_axis_name="s")


def _elu(x):
    return jnp.where(x > 0, x, jnp.exp(x) - 1.0)


def _mm1_body(x_ref, w_ref, as_ref, ad_ref, hch_ref, at_ref, dt_ref):
    c = pl.program_id(1)
    h = jnp.dot(x_ref[...], w_ref[0], preferred_element_type=jnp.float32)
    hch_ref[0] = h
    pa = jnp.sum(h * as_ref[0], axis=-1).reshape(1, 1, -1)
    pd = jnp.sum(h * ad_ref[0], axis=-1).reshape(1, 1, -1)

    @pl.when(c % CPH1 == 0)
    def _():
        at_ref[...] = pa
        dt_ref[...] = pd

    @pl.when(c % CPH1 != 0)
    def _():
        at_ref[...] += pa
        dt_ref[...] += pd


def _k_mm1(xp, W1, a1s, a1d):
    nb = NP // 512
    return pl.pallas_call(
        _mm1_body,
        grid=(nb, NC1),
        in_specs=[
            pl.BlockSpec((512, DIM), lambda n, c: (n, 0)),
            pl.BlockSpec((1, DIM, CW1), lambda n, c: (c, 0, 0)),
            pl.BlockSpec((1, 1, CW1), lambda n, c: (c, 0, 0)),
            pl.BlockSpec((1, 1, CW1), lambda n, c: (c, 0, 0)),
        ],
        out_specs=[
            pl.BlockSpec((1, 512, CW1), lambda n, c: (c, n, 0)),
            pl.BlockSpec((1, 1, 512), lambda n, c: (c // CPH1, 0, n)),
            pl.BlockSpec((1, 1, 512), lambda n, c: (c // CPH1, 0, n)),
        ],
        out_shape=[
            jax.ShapeDtypeStruct((NC1, NP, CW1), jnp.float32),
            jax.ShapeDtypeStruct((HEADS, 1, NP), jnp.float32),
            jax.ShapeDtypeStruct((HEADS, 1, NP), jnp.float32),
        ],
    )(xp, W1, a1s, a1d)


def _mm2_body(acc_ref, b1_ref, w2_ref, as2_ref, ad2_ref,
              g0_ref, g1_ref, g2_ref, g3_ref, at_ref, dt_ref):
    c = pl.program_id(1)
    blk = acc_ref[0]
    num = blk[:, :CW1]
    den = jnp.maximum(blk[:, CW1:CW1 + 1], 1e-30)
    h1 = _elu(num / den + b1_ref[0])
    g = jnp.dot(h1, w2_ref[0], preferred_element_type=jnp.float32)
    grefs = (g0_ref, g1_ref, g2_ref, g3_ref)

    @pl.when(c == 0)
    def _():
        for q, gr in enumerate(grefs):
            gr[...] = g[:, q * CW2:(q + 1) * CW2]

    @pl.when(c != 0)
    def _():
        for q, gr in enumerate(grefs):
            gr[...] += g[:, q * CW2:(q + 1) * CW2]

    @pl.when(c == NC1 - 1)
    def _():
        z = jnp.concatenate([gr[...] for gr in grefs], axis=1)
        at_ref[...] = jnp.sum(z * as2_ref[...], axis=-1).reshape(1, -1)
        dt_ref[...] = jnp.sum(z * ad2_ref[...], axis=-1).reshape(1, -1)


def _k_mm2(acc1, b1r, W2r, a2s, a2d):
    nb = NP // 512
    return pl.pallas_call(
        _mm2_body,
        grid=(nb, NC1),
        in_specs=[
            pl.BlockSpec((1, 512, RW1), lambda n, c: (c, n, 0)),
            pl.BlockSpec((1, 1, CW1), lambda n, c: (c, 0, 0)),
            pl.BlockSpec((1, CW1, EDIM), lambda n, c: (c, 0, 0)),
            pl.BlockSpec((1, EDIM), lambda n, c: (0, 0)),
            pl.BlockSpec((1, EDIM), lambda n, c: (0, 0)),
        ],
        out_specs=(
            [pl.BlockSpec((512, CW2), lambda n, c: (n, 0))] * NC2
            + [pl.BlockSpec((1, 512), lambda n, c: (0, n))] * 2
        ),
        out_shape=(
            [jax.ShapeDtypeStruct((NP, CW2), jnp.float32)] * NC2
            + [jax.ShapeDtypeStruct((1, NP), jnp.float32)] * 2
        ),
    )(acc1, b1r, W2r, a2s, a2d)


def _fin_body(acc_ref, b2_ref, wc_ref, bc_ref, z_ref, lg_ref):
    parts = []
    for q in range(NC2):
        blk = acc_ref[q]
        parts.append(blk[:, :CW2] / jnp.maximum(blk[:, CW2:CW2 + 1], 1e-30))
    z = _elu(jnp.concatenate(parts, axis=1) + b2_ref[...])
    z_ref[...] = z
    lg_ref[...] = (jnp.dot(z, wc_ref[...], preferred_element_type=jnp.float32)
                   + bc_ref[...])


def _k_fin(acc2, b2r, Wcp, bcp):
    nb = NP // 512
    return pl.pallas_call(
        _fin_body,
        grid=(nb,),
        in_specs=[
            pl.BlockSpec((NC2, 512, RW2), lambda n: (0, n, 0)),
            pl.BlockSpec((1, EDIM), lambda n: (0, 0)),
            pl.BlockSpec((EDIM, 128), lambda n: (0, 0)),
            pl.BlockSpec((1, 128), lambda n: (0, 0)),
        ],
        out_specs=[
            pl.BlockSpec((512, EDIM), lambda n: (n, 0)),
            pl.BlockSpec((512, 128), lambda n: (n, 0)),
        ],
        out_shape=[
            jax.ShapeDtypeStruct((NP, EDIM), jnp.float32),
            jax.ShapeDtypeStruct((NP, 128), jnp.float32),
        ],
    )(acc2, b2r, Wcp, bcp)


def _wk_body(nheads, et, m, at_hbm, dt_hbm, src_hbm, dst_hbm, w_hbm,
             atv, dtv, srcv, dstv, wv, sem):
    wid = lax.axis_index("s") * NSC + lax.axis_index("c")
    base = wid * m
    pltpu.sync_copy(at_hbm, atv)
    pltpu.sync_copy(dt_hbm, dtv)
    pltpu.sync_copy(src_hbm.at[pl.ds(base, m)], srcv)
    pltpu.sync_copy(dst_hbm.at[pl.ds(base, m)], dstv)

    def body(g, _):
        s16 = srcv[pl.ds(g * L, L)]
        d16 = dstv[pl.ds(g * L, L)]
        eid = base + g * L + lax.iota(jnp.int32, L)
        ok = eid < et
        for h in range(nheads):
            a = plsc.load_gather(atv, [s16 + h * NP])
            b = plsc.load_gather(dtv, [d16 + h * NP])
            al = a + b
            al = jnp.where(al > 0, al, NEG * al)
            w = jnp.where(ok, jnp.exp(al), 0.0)
            wv[h, pl.ds(g * L, L)] = w
        return ()

    lax.fori_loop(0, m // L, body, (), unroll=4)
    for h in range(nheads):
        pltpu.sync_copy(wv.at[h], w_hbm.at[h, pl.ds(base, m)])


def _k_w(nheads, et, et_pad, at_flat, dt_flat, srcs, dsts):
    m = et_pad // NW
    kfn = functools.partial(
        pl.kernel,
        mesh=_mesh,
        compiler_params=pltpu.CompilerParams(
            needs_layout_passes=False, use_tc_tiling_on_sc=False),
        out_type=jax.ShapeDtypeStruct((nheads, et_pad), jnp.float32),
        scratch_types=[
            pltpu.VMEM((nheads * NP,), jnp.float32),
            pltpu.VMEM((nheads * NP,), jnp.float32),
            pltpu.VMEM((m,), jnp.int32),
            pltpu.VMEM((m,), jnp.int32),
            pltpu.VMEM((nheads, m), jnp.float32),
            pltpu.SemaphoreType.DMA,
        ],
    )(functools.partial(_wk_body, nheads, et, m))
    return kfn(at_flat, dt_flat, srcs, dsts)


def _agg_body(ncpc, cph, nheads, cw, rw, et_pad, tbl_hbm, src_hbm, dst_hbm,
              w_hbm, out_hbm, srcall, dstall, wall, idx2, dst2, row2, staged2,
              zv, acc_sh, semg0, semg1, sems0, sems1):
    core = lax.axis_index("c")
    sid = lax.axis_index("s")
    mt = et_pad // NTEC
    nbat = mt // EB
    rows_per_tec = NP // NTEC
    semg = (semg0, semg1)
    sems = (sems0, sems1)

    pltpu.sync_copy(src_hbm.at[pl.ds(sid * mt, mt)], srcall)
    pltpu.sync_copy(dst_hbm.at[pl.ds(sid * mt, mt)], dstall)

    def z0body(i, _):
        for q in range(rw // L):
            zv[i, pl.ds(q * L, L)] = jnp.zeros((L,), jnp.float32)
            staged2[0, i, pl.ds(q * L, L)] = jnp.zeros((L,), jnp.float32)
            staged2[1, i, pl.ds(q * L, L)] = jnp.zeros((L,), jnp.float32)
        return ()
    lax.fori_loop(0, EB, z0body, ())

    def build_gather_idx(b, t, cidx):
        def ibody(g, _):
            idx2[b, pl.ds(g * L, L)] = (
                srcall[pl.ds(t * EB + g * L, L)] + cidx * NP)
            return ()
        lax.fori_loop(0, EB // L, ibody, (), unroll=8)

    for j in range(ncpc):
        cidx = core * ncpc + j
        head = cidx // cph
        pltpu.sync_copy(w_hbm.at[head, pl.ds(sid * mt, mt)], wall)

        def zbody(r, _):
            pltpu.sync_copy(
                zv, acc_sh.at[pl.ds(sid * rows_per_tec + r * EB, EB)])
            return ()
        lax.fori_loop(0, rows_per_tec // EB, zbody, ())
        plsc.subcore_barrier()

        build_gather_idx(0, 0, cidx)
        pltpu.async_copy(tbl_hbm.at[idx2.at[0]], row2.at[0], semg[0])

        def kbody(k, _):
            for b in range(2):
                t = 2 * k + b
                nb = 1 - b

                @pl.when(t + 1 < nbat)
                def _():
                    build_gather_idx(nb, t + 1, cidx)
                    pltpu.async_copy(
                        tbl_hbm.at[idx2.at[nb]], row2.at[nb], semg[nb])

                pltpu.make_async_copy(
                    tbl_hbm.at[idx2.at[b]], row2.at[b], semg[b]).wait()

                @pl.when(t >= 2)
                def _():
                    pltpu.make_async_copy(
                        staged2.at[b], acc_sh.at[dst2.at[b]], sems[b]).wait()

                rowv = row2.at[b]
                stagedv = staged2.at[b]

                def gbody(g, _):
                    w16 = wall[pl.ds(t * EB + g * L, L)]
                    dst2[b, pl.ds(g * L, L)] = dstall[pl.ds(t * EB + g * L, L)]
                    for lane in range(L):
                        i = g * L + lane
                        sp = plsc.load_gather(
                            wall,
                            [jnp.full((L,), t * EB + i, jnp.int32)])
                        for q in range(cw // L):
                            stagedv[i, pl.ds(q * L, L)] = (
                                rowv[i, pl.ds(q * L, L)] * sp)
                    rows16 = lax.iota(jnp.int32, L) + g * L
                    plsc.store_scatter(
                        stagedv, [rows16, jnp.full((L,), cw, jnp.int32)], w16)
                    return ()
                lax.fori_loop(0, EB // L, gbody, ())

                pltpu.async_copy(
                    staged2.at[b], acc_sh.at[dst2.at[b]], sems[b], add=True)
            return ()

        lax.fori_loop(0, nbat // 2, kbody, ())
        for b in range(2):
            pltpu.make_async_copy(
                staged2.at[b], acc_sh.at[dst2.at[b]], sems[b]).wait()
        plsc.subcore_barrier()
        pltpu.sync_copy(
            acc_sh.at[pl.ds(sid * rows_per_tec, rows_per_tec)],
            out_hbm.at[cidx].at[pl.ds(sid * rows_per_tec, rows_per_tec)])
        plsc.subcore_barrier()


def _k_agg(nchunks, nheads, cw, rw, et_pad, tbl, srcs, dsts, w):
    ncpc = nchunks // NSC
    cph = nchunks // nheads
    mt = et_pad // NTEC
    kfn = functools.partial(
        pl.kernel,
        mesh=_mesh,
        compiler_params=pltpu.CompilerParams(
            needs_layout_passes=False, use_tc_tiling_on_sc=False),
        out_type=jax.ShapeDtypeStruct((nchunks, NP, rw), jnp.float32),
        scratch_types=[
            pltpu.VMEM((mt,), jnp.int32),
            pltpu.VMEM((mt,), jnp.int32),
            pltpu.VMEM((mt,), jnp.float32),
            pltpu.VMEM((2, EB), jnp.int32),
            pltpu.VMEM((2, EB), jnp.int32),
            pltpu.VMEM((2, EB, cw), jnp.float32),
            pltpu.VMEM((2, EB, rw), jnp.float32),
            pltpu.VMEM((EB, rw), jnp.float32),
            pltpu.VMEM_SHARED((NP, rw), jnp.float32),
            pltpu.SemaphoreType.DMA,
            pltpu.SemaphoreType.DMA,
            pltpu.SemaphoreType.DMA,
            pltpu.SemaphoreType.DMA,
        ],
    )(functools.partial(_agg_body, ncpc, cph, nheads, cw, rw, et_pad))
    return kfn(tbl, srcs, dsts, w)


def _pred_body(pq_pad, z_hbm, i0_hbm, i1_hbm, out_hbm,
               i0v, i1v, av, bv, resv, sem):
    wid = lax.axis_index("s") * NSC + lax.axis_index("c")
    mp = pq_pad // NW
    base = wid * mp
    nbat = mp // EB

    def bbody(t, _):
        pbase = base + t * EB
        pltpu.sync_copy(i0_hbm.at[pl.ds(pbase, EB)], i0v)
        pltpu.sync_copy(i1_hbm.at[pl.ds(pbase, EB)], i1v)
        pltpu.async_copy(z_hbm.at[i0v], av, sem).wait()
        pltpu.async_copy(z_hbm.at[i1v], bv, sem).wait()

        lanes = lax.iota(jnp.int32, L)

        def gbody(g, _):
            res = jnp.zeros((L,), jnp.float32)
            for lane in range(L):
                i = g * L + lane
                acc = av[i, pl.ds(0, L)] * bv[i, pl.ds(0, L)]
                for q in range(1, EDIM // L):
                    acc = acc + av[i, pl.ds(q * L, L)] * bv[i, pl.ds(q * L, L)]
                s = jnp.sum(acc, axis=0)
                res = jnp.where(lanes == lane, s, res)
            resv[pl.ds(g * L, L)] = 1.0 / (1.0 + jnp.exp(-res))
            return ()
        lax.fori_loop(0, EB // L, gbody, ())

        pltpu.sync_copy(resv, out_hbm.at[pl.ds(pbase, EB)])
        return ()

    lax.fori_loop(0, nbat, bbody, ())


def _k_pred(pq_pad, z, i0, i1):
    kfn = functools.partial(
        pl.kernel,
        mesh=_mesh,
        compiler_params=pltpu.CompilerParams(
            needs_layout_passes=False, use_tc_tiling_on_sc=False),
        out_type=jax.ShapeDtypeStruct((pq_pad,), jnp.float32),
        scratch_types=[
            pltpu.VMEM((EB,), jnp.int32),
            pltpu.VMEM((EB,), jnp.int32),
            pltpu.VMEM((EB, EDIM), jnp.float32),
            pltpu.VMEM((EB, EDIM), jnp.float32),
            pltpu.VMEM((EB,), jnp.float32),
            pltpu.SemaphoreType.DMA,
        ],
    )(functools.partial(_pred_body, pq_pad))
    return kfn(z, i0, i1)


def kernel(x, e, p, n, W1, a_s1, a_d1, b1, W2, a_s2, a_d2, b2, Wc, bc):
    E = e.shape[1]
    P = p.shape[1]
    ET = E + N
    ET_pad = ((ET + NW * EB - 1) // (NW * EB)) * (NW * EB)
    PQ = 2 * P
    PQ_pad = ((PQ + NW * EB - 1) // (NW * EB)) * (NW * EB)

    xp = jnp.pad(x, ((0, NP - N), (0, 0)))
    loop = jnp.arange(N, dtype=e.dtype)
    ei = jnp.concatenate(
        [e, jnp.stack([loop, loop]),
         jnp.zeros((2, ET_pad - ET), e.dtype)], axis=1).astype(jnp.int32)
    srcs, dsts = ei[0], ei[1]
    pq = jnp.concatenate(
        [p, n, jnp.zeros((2, PQ_pad - PQ), p.dtype)], axis=1).astype(jnp.int32)
    a1s = a_s1.reshape(NC1, 1, CW1)
    a1d = a_d1.reshape(NC1, 1, CW1)
    b1r = b1.reshape(NC1, 1, CW1)
    W1r = W1.reshape(DIM, NC1, CW1).transpose(1, 0, 2)
    W2r = W2.reshape(NC1, CW1, EDIM)
    a2s = a_s2.reshape(1, EDIM)
    a2d = a_d2.reshape(1, EDIM)
    b2r = b2.reshape(1, EDIM)
    Wcp = jnp.pad(Wc, ((0, 0), (0, 128 - Wc.shape[1])))
    bcp = jnp.pad(bc, (0, 128 - bc.shape[0])).reshape(1, 128)

    hch, at1, dt1 = _k_mm1(xp, W1r, a1s, a1d)
    w1 = _k_w(HEADS, ET, ET_pad, at1.reshape(-1), dt1.reshape(-1), srcs, dsts)
    acc1 = _k_agg(NC1, HEADS, CW1, RW1, ET_pad,
                  hch.reshape(NC1 * NP, CW1), srcs, dsts, w1)

    g0, g1, g2, g3, at2, dt2 = _k_mm2(acc1, b1r, W2r, a2s, a2d)
    w2 = _k_w(1, ET, ET_pad, at2.reshape(-1), dt2.reshape(-1), srcs, dsts)
    g2s = jnp.concatenate([g0, g1, g2, g3], axis=0)
    acc2 = _k_agg(NC2, 1, CW2, RW2, ET_pad, g2s, srcs, dsts, w2)

    zf, lg = _k_fin(acc2, b2r, Wcp, bcp)
    preds = _k_pred(PQ_pad, zf, pq[0], pq[1])

    z = zf[:N]
    logits = lg[:N, :Wc.shape[1]]
    return (z, logits, preds[:PQ])


# speedup vs baseline: 10.8092x; 1.2666x over previous
import functools

import jax
import jax.numpy as jnp
import numpy as np
from jax import lax
from jax.experimental import pallas as pl
from jax.experimental.pallas import tpu as pltpu
from jax.experimental.pallas import tpu_sc as plsc

HEADS = 4
H = 256
EDIM = 128
NEG = 0.2

N = 10000
NP = 10240
DIM = 128
NC1 = 16
CW1 = 64
RW1 = 80
CPH1 = NC1 // HEADS
NC2 = 4
CW2 = 32
RW2 = 48

NSC = 2
NTEC = 16
NW = NSC * NTEC
L = 16

EB = 128

_mesh = plsc.VectorSubcoreMesh(core_axis_name="c", subcore_axis_name="s")


def _interleave_perm(width):
    p = []
    for q in range(width // 32):
        p.extend(range(32 * q, 32 * q + 32, 2))
        p.extend(range(32 * q + 1, 32 * q + 32, 2))
    return np.asarray(p, np.int32)


_P1 = np.concatenate([64 * c + _interleave_perm(64) for c in range(NC1)])
_P2 = np.concatenate([32 * c + _interleave_perm(32) for c in range(NC2)])
_P2_INV = np.argsort(_P2)


def _elu(x):
    return jnp.where(x > 0, x, jnp.exp(x) - 1.0)


def _mm1_body(x_ref, w_ref, as_ref, ad_ref, hch_ref, at_ref, dt_ref):
    c = pl.program_id(1)
    h = jnp.dot(x_ref[...], w_ref[0], preferred_element_type=jnp.float32)
    hch_ref[0] = h.astype(jnp.bfloat16)
    pa = jnp.sum(h * as_ref[0], axis=-1).reshape(1, 1, -1)
    pd = jnp.sum(h * ad_ref[0], axis=-1).reshape(1, 1, -1)

    @pl.when(c % CPH1 == 0)
    def _():
        at_ref[...] = pa
        dt_ref[...] = pd

    @pl.when(c % CPH1 != 0)
    def _():
        at_ref[...] += pa
        dt_ref[...] += pd


def _k_mm1(xp, W1, a1s, a1d):
    nb = NP // 512
    return pl.pallas_call(
        _mm1_body,
        grid=(nb, NC1),
        in_specs=[
            pl.BlockSpec((512, DIM), lambda n, c: (n, 0)),
            pl.BlockSpec((1, DIM, CW1), lambda n, c: (c, 0, 0)),
            pl.BlockSpec((1, 1, CW1), lambda n, c: (c, 0, 0)),
            pl.BlockSpec((1, 1, CW1), lambda n, c: (c, 0, 0)),
        ],
        out_specs=[
            pl.BlockSpec((1, 512, CW1), lambda n, c: (c, n, 0)),
            pl.BlockSpec((1, 1, 512), lambda n, c: (c // CPH1, 0, n)),
            pl.BlockSpec((1, 1, 512), lambda n, c: (c // CPH1, 0, n)),
        ],
        out_shape=[
            jax.ShapeDtypeStruct((NC1, NP, CW1), jnp.bfloat16),
            jax.ShapeDtypeStruct((HEADS, 1, NP), jnp.float32),
            jax.ShapeDtypeStruct((HEADS, 1, NP), jnp.float32),
        ],
    )(xp, W1, a1s, a1d)


def _mm2_body(acc_ref, b1_ref, w2_ref, as2_ref, ad2_ref,
              g0_ref, g1_ref, g2_ref, g3_ref,
              gb0_ref, gb1_ref, gb2_ref, gb3_ref, at_ref, dt_ref):
    c = pl.program_id(1)
    blk = acc_ref[0]
    num = blk[:, :CW1]
    den = jnp.maximum(blk[:, CW1:CW1 + 1], 1e-30)
    h1 = _elu(num / den + b1_ref[0])
    g = jnp.dot(h1, w2_ref[0], preferred_element_type=jnp.float32)
    grefs = (g0_ref, g1_ref, g2_ref, g3_ref)
    gbrefs = (gb0_ref, gb1_ref, gb2_ref, gb3_ref)

    @pl.when(c == 0)
    def _():
        for q, gr in enumerate(grefs):
            gr[...] = g[:, q * CW2:(q + 1) * CW2]

    @pl.when(c != 0)
    def _():
        for q, gr in enumerate(grefs):
            gr[...] += g[:, q * CW2:(q + 1) * CW2]

    @pl.when(c == NC1 - 1)
    def _():
        for gr, gbr in zip(grefs, gbrefs):
            gbr[...] = gr[...].astype(jnp.bfloat16)
        z = jnp.concatenate([gr[...] for gr in grefs], axis=1)
        at_ref[...] = jnp.sum(z * as2_ref[...], axis=-1).reshape(1, -1)
        dt_ref[...] = jnp.sum(z * ad2_ref[...], axis=-1).reshape(1, -1)


def _k_mm2(acc1, b1r, W2r, a2s, a2d):
    nb = NP // 512
    return pl.pallas_call(
        _mm2_body,
        grid=(nb, NC1),
        in_specs=[
            pl.BlockSpec((1, 512, RW1), lambda n, c: (c, n, 0)),
            pl.BlockSpec((1, 1, CW1), lambda n, c: (c, 0, 0)),
            pl.BlockSpec((1, CW1, EDIM), lambda n, c: (c, 0, 0)),
            pl.BlockSpec((1, EDIM), lambda n, c: (0, 0)),
            pl.BlockSpec((1, EDIM), lambda n, c: (0, 0)),
        ],
        out_specs=(
            [pl.BlockSpec((512, CW2), lambda n, c: (n, 0))] * (2 * NC2)
            + [pl.BlockSpec((1, 512), lambda n, c: (0, n))] * 2
        ),
        out_shape=(
            [jax.ShapeDtypeStruct((NP, CW2), jnp.float32)] * NC2
            + [jax.ShapeDtypeStruct((NP, CW2), jnp.bfloat16)] * NC2
            + [jax.ShapeDtypeStruct((1, NP), jnp.float32)] * 2
        ),
    )(acc1, b1r, W2r, a2s, a2d)


def _fin_body(acc_ref, b2_ref, wc_ref, bc_ref, z_ref, lg_ref):
    parts = []
    for q in range(NC2):
        blk = acc_ref[q]
        parts.append(blk[:, :CW2] / jnp.maximum(blk[:, CW2:CW2 + 1], 1e-30))
    z = _elu(jnp.concatenate(parts, axis=1) + b2_ref[...])
    z_ref[...] = z
    lg_ref[...] = (jnp.dot(z, wc_ref[...], preferred_element_type=jnp.float32)
                   + bc_ref[...])


def _k_fin(acc2, b2r, Wcp, bcp):
    nb = NP // 512
    return pl.pallas_call(
        _fin_body,
        grid=(nb,),
        in_specs=[
            pl.BlockSpec((NC2, 512, RW2), lambda n: (0, n, 0)),
            pl.BlockSpec((1, EDIM), lambda n: (0, 0)),
            pl.BlockSpec((EDIM, 128), lambda n: (0, 0)),
            pl.BlockSpec((1, 128), lambda n: (0, 0)),
        ],
        out_specs=[
            pl.BlockSpec((512, EDIM), lambda n: (n, 0)),
            pl.BlockSpec((512, 128), lambda n: (n, 0)),
        ],
        out_shape=[
            jax.ShapeDtypeStruct((NP, EDIM), jnp.float32),
            jax.ShapeDtypeStruct((NP, 128), jnp.float32),
        ],
    )(acc2, b2r, Wcp, bcp)


def _wk_body(nheads, et, m, at_hbm, dt_hbm, src_hbm, dst_hbm, w_hbm,
             atv, dtv, srcv, dstv, wv, sem):
    wid = lax.axis_index("s") * NSC + lax.axis_index("c")
    base = wid * m
    pltpu.sync_copy(at_hbm, atv)
    pltpu.sync_copy(dt_hbm, dtv)
    pltpu.sync_copy(src_hbm.at[pl.ds(base, m)], srcv)
    pltpu.sync_copy(dst_hbm.at[pl.ds(base, m)], dstv)

    def body(g, _):
        s16 = srcv[pl.ds(g * L, L)]
        d16 = dstv[pl.ds(g * L, L)]
        eid = base + g * L + lax.iota(jnp.int32, L)
        ok = eid < et
        for h in range(nheads):
            a = plsc.load_gather(atv, [s16 + h * NP])
            b = plsc.load_gather(dtv, [d16 + h * NP])
            al = a + b
            al = jnp.where(al > 0, al, NEG * al)
            w = jnp.where(ok, jnp.exp(al), 0.0)
            wv[h, pl.ds(g * L, L)] = w
        return ()

    lax.fori_loop(0, m // L, body, (), unroll=4)
    for h in range(nheads):
        pltpu.sync_copy(wv.at[h], w_hbm.at[h, pl.ds(base, m)])


def _k_w(nheads, et, et_pad, at_flat, dt_flat, srcs, dsts):
    m = et_pad // NW
    kfn = functools.partial(
        pl.kernel,
        mesh=_mesh,
        compiler_params=pltpu.CompilerParams(
            needs_layout_passes=False, use_tc_tiling_on_sc=False),
        out_type=jax.ShapeDtypeStruct((nheads, et_pad), jnp.float32),
        scratch_types=[
            pltpu.VMEM((nheads * NP,), jnp.float32),
            pltpu.VMEM((nheads * NP,), jnp.float32),
            pltpu.VMEM((m,), jnp.int32),
            pltpu.VMEM((m,), jnp.int32),
            pltpu.VMEM((nheads, m), jnp.float32),
            pltpu.SemaphoreType.DMA,
        ],
    )(functools.partial(_wk_body, nheads, et, m))
    return kfn(at_flat, dt_flat, srcs, dsts)


def _agg_body(ncpc, cph, nheads, cw, rw, et_pad, tbl_hbm, src_hbm, dst_hbm,
              w_hbm, out_hbm, srcall, dstall, wall, idx2, dst2, row2, staged2,
              zv, acc_sh, semg0, semg1, sems0, sems1):
    core = lax.axis_index("c")
    sid = lax.axis_index("s")
    mt = et_pad // NTEC
    nbat = mt // EB
    rows_per_tec = NP // NTEC
    semg = (semg0, semg1)
    sems = (sems0, sems1)

    pltpu.sync_copy(src_hbm.at[pl.ds(sid * mt, mt)], srcall)
    pltpu.sync_copy(dst_hbm.at[pl.ds(sid * mt, mt)], dstall)

    def z0body(i, _):
        for q in range(rw // L):
            zv[i, pl.ds(q * L, L)] = jnp.zeros((L,), jnp.float32)
            staged2[0, i, pl.ds(q * L, L)] = jnp.zeros((L,), jnp.float32)
            staged2[1, i, pl.ds(q * L, L)] = jnp.zeros((L,), jnp.float32)
        return ()
    lax.fori_loop(0, EB, z0body, ())

    def build_gather_idx(b, t, cidx):
        def ibody(g, _):
            idx2[b, pl.ds(g * L, L)] = (
                srcall[pl.ds(t * EB + g * L, L)] + cidx * NP)
            return ()
        lax.fori_loop(0, EB // L, ibody, (), unroll=8)

    for j in range(ncpc):
        cidx = core * ncpc + j
        head = cidx // cph
        pltpu.sync_copy(w_hbm.at[head, pl.ds(sid * mt, mt)], wall)

        def zbody(r, _):
            pltpu.sync_copy(
                zv, acc_sh.at[pl.ds(sid * rows_per_tec + r * EB, EB)])
            return ()
        lax.fori_loop(0, rows_per_tec // EB, zbody, ())
        plsc.subcore_barrier()

        build_gather_idx(0, 0, cidx)
        pltpu.async_copy(tbl_hbm.at[idx2.at[0]], row2.at[0], semg[0])

        def kbody(k, _):
            for b in range(2):
                t = 2 * k + b
                nb = 1 - b

                @pl.when(t + 1 < nbat)
                def _():
                    build_gather_idx(nb, t + 1, cidx)
                    pltpu.async_copy(
                        tbl_hbm.at[idx2.at[nb]], row2.at[nb], semg[nb])

                pltpu.make_async_copy(
                    tbl_hbm.at[idx2.at[b]], row2.at[b], semg[b]).wait()

                @pl.when(t >= 2)
                def _():
                    pltpu.make_async_copy(
                        staged2.at[b], acc_sh.at[dst2.at[b]], sems[b]).wait()

                rowv = row2.at[b]
                stagedv = staged2.at[b]

                def gbody(g, _):
                    w16 = wall[pl.ds(t * EB + g * L, L)]
                    dst2[b, pl.ds(g * L, L)] = dstall[pl.ds(t * EB + g * L, L)]
                    for lane in range(L):
                        i = g * L + lane
                        sp = plsc.load_gather(
                            wall,
                            [jnp.full((L,), t * EB + i, jnp.int32)])
                        for q in range(cw // 32):
                            v = rowv[i, pl.ds(q * 32, 32)]
                            lo, hi = plsc.unpack(
                                v, format=plsc.PackFormat.INTERLEAVED)
                            stagedv[i, pl.ds(q * 32, L)] = lo * sp
                            stagedv[i, pl.ds(q * 32 + L, L)] = hi * sp
                    rows16 = lax.iota(jnp.int32, L) + g * L
                    plsc.store_scatter(
                        stagedv, [rows16, jnp.full((L,), cw, jnp.int32)], w16)
                    return ()
                lax.fori_loop(0, EB // L, gbody, ())

                pltpu.async_copy(
                    staged2.at[b], acc_sh.at[dst2.at[b]], sems[b], add=True)
            return ()

        lax.fori_loop(0, nbat // 2, kbody, ())
        for b in range(2):
            pltpu.make_async_copy(
                staged2.at[b], acc_sh.at[dst2.at[b]], sems[b]).wait()
        plsc.subcore_barrier()
        pltpu.sync_copy(
            acc_sh.at[pl.ds(sid * rows_per_tec, rows_per_tec)],
            out_hbm.at[cidx].at[pl.ds(sid * rows_per_tec, rows_per_tec)])
        plsc.subcore_barrier()


def _k_agg(nchunks, nheads, cw, rw, et_pad, tbl, srcs, dsts, w):
    ncpc = nchunks // NSC
    cph = nchunks // nheads
    mt = et_pad // NTEC
    kfn = functools.partial(
        pl.kernel,
        mesh=_mesh,
        compiler_params=pltpu.CompilerParams(
            needs_layout_passes=False, use_tc_tiling_on_sc=False),
        out_type=jax.ShapeDtypeStruct((nchunks, NP, rw), jnp.float32),
        scratch_types=[
            pltpu.VMEM((mt,), jnp.int32),
            pltpu.VMEM((mt,), jnp.int32),
            pltpu.VMEM((mt,), jnp.float32),
            pltpu.VMEM((2, EB), jnp.int32),
            pltpu.VMEM((2, EB), jnp.int32),
            pltpu.VMEM((2, EB, cw), jnp.bfloat16),
            pltpu.VMEM((2, EB, rw), jnp.float32),
            pltpu.VMEM((EB, rw), jnp.float32),
            pltpu.VMEM_SHARED((NP, rw), jnp.float32),
            pltpu.SemaphoreType.DMA,
            pltpu.SemaphoreType.DMA,
            pltpu.SemaphoreType.DMA,
            pltpu.SemaphoreType.DMA,
        ],
    )(functools.partial(_agg_body, ncpc, cph, nheads, cw, rw, et_pad))
    return kfn(tbl, srcs, dsts, w)


def _pred_body(pq_pad, z_hbm, i0_hbm, i1_hbm, out_hbm,
               i0v, i1v, av, bv, resv, sem):
    wid = lax.axis_index("s") * NSC + lax.axis_index("c")
    mp = pq_pad // NW
    base = wid * mp
    nbat = mp // EB

    def bbody(t, _):
        pbase = base + t * EB
        pltpu.sync_copy(i0_hbm.at[pl.ds(pbase, EB)], i0v)
        pltpu.sync_copy(i1_hbm.at[pl.ds(pbase, EB)], i1v)
        pltpu.async_copy(z_hbm.at[i0v], av, sem).wait()
        pltpu.async_copy(z_hbm.at[i1v], bv, sem).wait()

        lanes = lax.iota(jnp.int32, L)

        def gbody(g, _):
            res = jnp.zeros((L,), jnp.float32)
            for lane in range(L):
                i = g * L + lane
                acc = av[i, pl.ds(0, L)] * bv[i, pl.ds(0, L)]
                for q in range(1, EDIM // L):
                    acc = acc + av[i, pl.ds(q * L, L)] * bv[i, pl.ds(q * L, L)]
                s = jnp.sum(acc, axis=0)
                res = jnp.where(lanes == lane, s, res)
            resv[pl.ds(g * L, L)] = 1.0 / (1.0 + jnp.exp(-res))
            return ()
        lax.fori_loop(0, EB // L, gbody, ())

        pltpu.sync_copy(resv, out_hbm.at[pl.ds(pbase, EB)])
        return ()

    lax.fori_loop(0, nbat, bbody, ())


def _k_pred(pq_pad, z, i0, i1):
    kfn = functools.partial(
        pl.kernel,
        mesh=_mesh,
        compiler_params=pltpu.CompilerParams(
            needs_layout_passes=False, use_tc_tiling_on_sc=False),
        out_type=jax.ShapeDtypeStruct((pq_pad,), jnp.float32),
        scratch_types=[
            pltpu.VMEM((EB,), jnp.int32),
            pltpu.VMEM((EB,), jnp.int32),
            pltpu.VMEM((EB, EDIM), jnp.float32),
            pltpu.VMEM((EB, EDIM), jnp.float32),
            pltpu.VMEM((EB,), jnp.float32),
            pltpu.SemaphoreType.DMA,
        ],
    )(functools.partial(_pred_body, pq_pad))
    return kfn(z, i0, i1)


def kernel(x, e, p, n, W1, a_s1, a_d1, b1, W2, a_s2, a_d2, b2, Wc, bc):
    E = e.shape[1]
    P = p.shape[1]
    ET = E + N
    ET_pad = ((ET + NW * EB - 1) // (NW * EB)) * (NW * EB)
    PQ = 2 * P
    PQ_pad = ((PQ + NW * EB - 1) // (NW * EB)) * (NW * EB)

    xp = jnp.pad(x, ((0, NP - N), (0, 0)))
    loop = jnp.arange(N, dtype=e.dtype)
    ei = jnp.concatenate(
        [e, jnp.stack([loop, loop]),
         jnp.zeros((2, ET_pad - ET), e.dtype)], axis=1).astype(jnp.int32)
    srcs, dsts = ei[0], ei[1]
    pq = jnp.concatenate(
        [p, n, jnp.zeros((2, PQ_pad - PQ), p.dtype)], axis=1).astype(jnp.int32)
    a1s = a_s1.reshape(NC1, 1, CW1)
    a1d = a_d1.reshape(NC1, 1, CW1)
    p1 = jnp.asarray(_P1)
    p2 = jnp.asarray(_P2)
    b1r = b1[p1].reshape(NC1, 1, CW1)
    W1r = W1.reshape(DIM, NC1, CW1).transpose(1, 0, 2)
    W2r = W2[p1, :].reshape(NC1, CW1, EDIM)
    a2s = a_s2.reshape(1, EDIM)
    a2d = a_d2.reshape(1, EDIM)
    b2r = b2[p2].reshape(1, EDIM)
    Wcp = jnp.pad(Wc[p2, :], ((0, 0), (0, 128 - Wc.shape[1])))
    bcp = jnp.pad(bc, (0, 128 - bc.shape[0])).reshape(1, 128)

    hch, at1, dt1 = _k_mm1(xp, W1r, a1s, a1d)
    w1 = _k_w(HEADS, ET, ET_pad, at1.reshape(-1), dt1.reshape(-1), srcs, dsts)
    acc1 = _k_agg(NC1, HEADS, CW1, RW1, ET_pad,
                  hch.reshape(NC1 * NP, CW1), srcs, dsts, w1)

    (g0, g1, g2, g3, gb0, gb1, gb2, gb3, at2, dt2) = _k_mm2(
        acc1, b1r, W2r, a2s, a2d)
    del g0, g1, g2, g3
    w2 = _k_w(1, ET, ET_pad, at2.reshape(-1), dt2.reshape(-1), srcs, dsts)
    g2s = jnp.concatenate([gb0, gb1, gb2, gb3], axis=0)
    acc2 = _k_agg(NC2, 1, CW2, RW2, ET_pad, g2s, srcs, dsts, w2)

    zf, lg = _k_fin(acc2, b2r, Wcp, bcp)
    preds = _k_pred(PQ_pad, zf, pq[0], pq[1])

    z = zf[:, jnp.asarray(_P2_INV)][:N]
    logits = lg[:N, :Wc.shape[1]]
    return (z, logits, preds[:PQ])
